# baseline (speedup 1.0000x reference)
import jax
import jax.numpy as jnp
from jax.experimental import pallas as pl

_S_BLK = 256


def _add_pe_body(x_ref, pe_ref, o_ref):
    o_ref[...] = x_ref[...] + pe_ref[...][None, :, :]


def kernel(x, pe):
    batch, seq_len, d_model = x.shape
    pe = pe[:seq_len]
    grid = (seq_len // _S_BLK,)
    return pl.pallas_call(
        _add_pe_body,
        grid=grid,
        in_specs=[
            pl.BlockSpec((batch, _S_BLK, d_model), lambda i: (0, i, 0)),
            pl.BlockSpec((_S_BLK, d_model), lambda i: (i, 0)),
        ],
        out_specs=pl.BlockSpec((batch, _S_BLK, d_model), lambda i: (0, i, 0)),
        out_shape=jax.ShapeDtypeStruct(x.shape, x.dtype),
    )(x, pe)


# speedup vs baseline: 1.0193x; 1.0193x over previous
import jax
import jax.numpy as jnp
from jax.experimental import pallas as pl

_S_BLK = 2048


def _add_pe_body(x_ref, pe_ref, o_ref):
    o_ref[...] = x_ref[...] + pe_ref[...][None, :, :]


def kernel(x, pe):
    batch, seq_len, d_model = x.shape
    pe = pe[:seq_len]
    grid = (seq_len // _S_BLK, batch)
    return pl.pallas_call(
        _add_pe_body,
        grid=grid,
        in_specs=[
            pl.BlockSpec((1, _S_BLK, d_model), lambda j, b: (b, j, 0)),
            pl.BlockSpec((_S_BLK, d_model), lambda j, b: (j, 0)),
        ],
        out_specs=pl.BlockSpec((1, _S_BLK, d_model), lambda j, b: (b, j, 0)),
        out_shape=jax.ShapeDtypeStruct(x.shape, x.dtype),
    )(x, pe)


# speedup vs baseline: 1.0201x; 1.0007x over previous
import jax
import jax.numpy as jnp
from jax.experimental import pallas as pl
from jax.experimental.pallas import tpu as pltpu

_S_BLK = 2048


def _add_pe_body(x_ref, pe_ref, o_ref):
    o_ref[...] = x_ref[...] + pe_ref[...][None, :, :]


def kernel(x, pe):
    batch, seq_len, d_model = x.shape
    pe = pe[:seq_len]
    grid = (seq_len // _S_BLK, batch)
    return pl.pallas_call(
        _add_pe_body,
        grid=grid,
        in_specs=[
            pl.BlockSpec((1, _S_BLK, d_model), lambda j, b: (b, j, 0)),
            pl.BlockSpec((_S_BLK, d_model), lambda j, b: (j, 0)),
        ],
        out_specs=pl.BlockSpec((1, _S_BLK, d_model), lambda j, b: (b, j, 0)),
        out_shape=jax.ShapeDtypeStruct(x.shape, x.dtype),
        compiler_params=pltpu.CompilerParams(
            dimension_semantics=("parallel", "parallel"),
        ),
    )(x, pe)
